# Initial kernel scaffold; baseline (speedup 1.0000x reference)
import functools

import jax
import jax.numpy as jnp
from jax import lax
from jax.experimental import pallas as pl
from jax.experimental.pallas import tpu as pltpu
from jax.experimental.pallas import tpu_sc as plsc

BATCH = 16384
HIST = 200
EMBED = 32
B = BATCH * HIST

_info = plsc.get_sparse_core_info()
NC, NS = _info.num_cores, _info.num_subcores
NW = NC * NS
BPW = B // NW
CHUNK = 2048
NCHUNK = BPW // CHUNK

_mesh = plsc.VectorSubcoreMesh(core_axis_name="c", subcore_axis_name="s")


@functools.partial(
    pl.kernel,
    mesh=_mesh,
    out_type=jax.ShapeDtypeStruct((B, EMBED), jnp.float32),
    scratch_types=[
        pltpu.VMEM((CHUNK,), jnp.int32),
        pltpu.VMEM((CHUNK, EMBED), jnp.float32),
        pltpu.SemaphoreType.DMA,
    ],
)
def _embed_lookup(idx_hbm, table_hbm, out_hbm, idx_v, rows_v, sem):
    wid = lax.axis_index("s") * NC + lax.axis_index("c")
    base = wid * BPW

    def body(i, carry):
        off = base + i * CHUNK
        pltpu.sync_copy(idx_hbm.at[pl.ds(off, CHUNK)], idx_v)
        pltpu.async_copy(table_hbm.at[idx_v], rows_v, sem).wait()
        pltpu.sync_copy(rows_v, out_hbm.at[pl.ds(off, CHUNK)])
        return carry

    lax.fori_loop(0, NCHUNK, body, 0)


def kernel(action, table):
    flat = action.reshape(B)
    out = _embed_lookup(flat, table)
    return out.reshape(BATCH, HIST, EMBED)


# speedup vs baseline: 4.9448x; 4.9448x over previous
import functools

import jax
import jax.numpy as jnp
from jax import lax
from jax.experimental import pallas as pl
from jax.experimental.pallas import tpu as pltpu
from jax.experimental.pallas import tpu_sc as plsc

BATCH = 16384
HIST = 200
EMBED = 32
B = BATCH * HIST
LANE = 128
ROWS = B // LANE

_info = plsc.get_sparse_core_info()
NC, NS = _info.num_cores, _info.num_subcores
NW = NC * NS
RPW = ROWS // NW
R = 16
NCHUNK = RPW // R

_mesh = plsc.VectorSubcoreMesh(core_axis_name="c", subcore_axis_name="s")


@functools.partial(
    pl.kernel,
    mesh=_mesh,
    out_type=jax.ShapeDtypeStruct((ROWS, LANE, EMBED), jnp.float32),
    compiler_params=pltpu.CompilerParams(use_tc_tiling_on_sc=False),
    scratch_types=[
        pltpu.VMEM((R, LANE), jnp.int32),
        pltpu.VMEM((R, LANE, EMBED), jnp.float32),
        pltpu.SemaphoreType.DMA,
    ],
)
def _embed_lookup(idx_hbm, table_hbm, out_hbm, idx_v, rows_v, sem):
    wid = lax.axis_index("s") * NC + lax.axis_index("c")
    base = wid * RPW

    def body(i, carry):
        off = base + i * R
        pltpu.sync_copy(idx_hbm.at[pl.ds(off, R)], idx_v)
        copies = [
            pltpu.async_copy(table_hbm.at[idx_v.at[j]], rows_v.at[j], sem)
            for j in range(R)
        ]
        for c in copies:
            c.wait()
        pltpu.sync_copy(rows_v, out_hbm.at[pl.ds(off, R)])
        return carry

    lax.fori_loop(0, NCHUNK, body, 0)


def kernel(action, table):
    idx = action.reshape(ROWS, LANE)
    out = _embed_lookup(idx, table)
    return out.reshape(BATCH, HIST, EMBED)


# speedup vs baseline: 5.0503x; 1.0213x over previous
import functools

import jax
import jax.numpy as jnp
from jax import lax
from jax.experimental import pallas as pl
from jax.experimental.pallas import tpu as pltpu
from jax.experimental.pallas import tpu_sc as plsc

BATCH = 16384
HIST = 200
EMBED = 32
B = BATCH * HIST
LANE = 128
ROWS = B // LANE

_info = plsc.get_sparse_core_info()
NC, NS = _info.num_cores, _info.num_subcores
NW = NC * NS
RPW = ROWS // NW
R = 8
NCHUNK = RPW // R
NPAIR = NCHUNK // 2

_mesh = plsc.VectorSubcoreMesh(core_axis_name="c", subcore_axis_name="s")


@functools.partial(
    pl.kernel,
    mesh=_mesh,
    out_type=jax.ShapeDtypeStruct((ROWS, LANE, EMBED), jnp.float32),
    compiler_params=pltpu.CompilerParams(use_tc_tiling_on_sc=False),
    scratch_types=[
        pltpu.VMEM((R, LANE), jnp.int32),
        pltpu.VMEM((R, LANE), jnp.int32),
        pltpu.VMEM((R, LANE, EMBED), jnp.float32),
        pltpu.VMEM((R, LANE, EMBED), jnp.float32),
        pltpu.SemaphoreType.DMA,
        pltpu.SemaphoreType.DMA,
        pltpu.SemaphoreType.DMA,
        pltpu.SemaphoreType.DMA,
    ],
)
def _embed_lookup(idx_hbm, table_hbm, out_hbm, idx0, idx1, rows0, rows1,
                  gsem0, gsem1, wsem0, wsem1):
    wid = lax.axis_index("s") * NC + lax.axis_index("c")
    base = wid * RPW
    idx = (idx0, idx1)
    rows = (rows0, rows1)
    gsem = (gsem0, gsem1)
    wsem = (wsem0, wsem1)

    def load_idx(b, off):
        pltpu.sync_copy(idx_hbm.at[pl.ds(off, R)], idx[b])

    def fire_gather(b):
        for j in range(R):
            pltpu.async_copy(table_hbm.at[idx[b].at[j]], rows[b].at[j],
                             gsem[b])

    def drain_gather(b):
        for j in range(R):
            pltpu.make_async_copy(table_hbm.at[idx[b].at[j]], rows[b].at[j],
                                  gsem[b]).wait()

    def fire_write(b, off):
        pltpu.async_copy(rows[b], out_hbm.at[pl.ds(off, R)], wsem[b])

    def drain_write(b):
        pltpu.make_async_copy(rows[b], out_hbm.at[pl.ds(0, R)],
                              wsem[b]).wait()

    load_idx(0, base)
    fire_gather(0)

    def body(g, carry):
        off0 = base + 2 * g * R

        load_idx(1, off0 + R)

        @pl.when(g > 0)
        def _():
            drain_write(1)

        fire_gather(1)
        drain_gather(0)
        fire_write(0, off0)

        @pl.when(g < NPAIR - 1)
        def _():
            load_idx(0, off0 + 2 * R)
            drain_write(0)
            fire_gather(0)

        drain_gather(1)
        fire_write(1, off0 + R)
        return carry

    lax.fori_loop(0, NPAIR, body, 0)
    drain_write(0)
    drain_write(1)


def kernel(action, table):
    idx = action.reshape(ROWS, LANE)
    out = _embed_lookup(idx, table)
    return out.reshape(BATCH, HIST, EMBED)
